# baseline (speedup 1.0000x reference)
import functools

import jax
import jax.numpy as jnp
from jax import lax
from jax.experimental import pallas as pl
from jax.experimental.pallas import tpu as pltpu
from jax.experimental.pallas import tpu_sc as plsc

N = 10000
E = 320000
D = 128
C = 50
NC = 2
NS = 16
NW = NC * NS
EPW = E // NW
K = 125
CH = EPW // K
CH2 = CH // 2
NP = 10240
RPT = NP // NS
NB = 10
BN = NP // NB

@functools.lru_cache(maxsize=None)
def _mesh():
    return plsc.VectorSubcoreMesh(core_axis_name="c", subcore_axis_name="s")


def _deg_body(src_i, dst_i, outs_h, outd_h, idx_s, idx_d, hist_s, hist_d):
    c = lax.axis_index("c")
    s = lax.axis_index("s")
    wid = c * NS + s
    pltpu.sync_copy(src_i.at[pl.ds(wid * EPW, EPW)], idx_s)
    pltpu.sync_copy(dst_i.at[pl.ds(wid * EPW, EPW)], idx_d)

    zeros16 = jnp.zeros((16,), jnp.float32)

    def z(i, carry):
        hist_s[pl.ds(i * 16, 16)] = zeros16
        hist_d[pl.ds(i * 16, 16)] = zeros16
        return carry

    lax.fori_loop(0, NP // 16, z, 0)

    ones16 = jnp.ones((16,), jnp.float32)

    def acc(i, carry):
        plsc.addupdate_scatter(hist_s, [idx_s[pl.ds(i * 16, 16)]], ones16)
        plsc.addupdate_scatter(hist_d, [idx_d[pl.ds(i * 16, 16)]], ones16)
        return carry

    lax.fori_loop(0, EPW // 16, acc, 0)
    pltpu.sync_copy(hist_s, outs_h.at[pl.ds(wid * NP, NP)])
    pltpu.sync_copy(hist_d, outd_h.at[pl.ds(wid * NP, NP)])


@functools.lru_cache(maxsize=None)
def _deg_call():
    return pl.kernel(
        _deg_body,
        out_type=(jax.ShapeDtypeStruct((NW * NP,), jnp.float32),
                  jax.ShapeDtypeStruct((NW * NP,), jnp.float32)),
        mesh=_mesh(),
        scratch_types=[
            pltpu.VMEM((EPW,), jnp.int32),
            pltpu.VMEM((EPW,), jnp.int32),
            pltpu.VMEM((NP,), jnp.float32),
            pltpu.VMEM((NP,), jnp.float32),
        ],
        compiler_params=pltpu.CompilerParams(needs_layout_passes=False),
    )


def _agg_body(h_h, src_i, dst_i, out_h, idx_s, idx_d, rows0, rows1,
              acc, semg0, semg1, sems0, sems1):
    c = lax.axis_index("c")
    s = lax.axis_index("s")
    wid = c * NS + s

    zf = jnp.zeros((16,), jnp.float32)

    def zz(i, carry):
        rows0[i // 8, pl.ds((i % 8) * 16, 16)] = zf
        return carry

    lax.fori_loop(0, 128 * 8, zz, 0)
    for t in range(RPT // 128):
        pltpu.sync_copy(rows0, acc.at[pl.ds(s * RPT + t * 128, 128)])
    plsc.subcore_barrier()

    g0 = rows0.at[pl.ds(0, K)]
    g1 = rows1.at[pl.ds(0, K)]

    def half(hf):
        pltpu.sync_copy(src_i.at[wid, pl.ds(hf * CH2, CH2)], idx_s)
        pltpu.sync_copy(dst_i.at[wid, pl.ds(hf * CH2, CH2)], idx_d)
        pltpu.async_copy(h_h.at[idx_s.at[0]], g0, semg0)

        def pair(i, carry):
            j = 2 * i
            pltpu.make_async_copy(h_h.at[idx_s.at[j]], g0, semg0).wait()
            pltpu.make_async_copy(g0, acc.at[idx_d.at[j]], sems0).start(add=True)

            @pl.when(i > 0)
            def _():
                pltpu.make_async_copy(g1, acc.at[idx_d.at[j - 1]], sems1).wait()

            pltpu.async_copy(h_h.at[idx_s.at[j + 1]], g1, semg1)
            pltpu.make_async_copy(h_h.at[idx_s.at[j + 1]], g1, semg1).wait()
            pltpu.make_async_copy(g1, acc.at[idx_d.at[j + 1]], sems1).start(add=True)
            pltpu.make_async_copy(g0, acc.at[idx_d.at[j]], sems0).wait()

            @pl.when(j + 2 < CH2)
            def _():
                pltpu.async_copy(h_h.at[idx_s.at[j + 2]], g0, semg0)

            return carry

        lax.fori_loop(0, CH2 // 2, pair, 0)
        pltpu.make_async_copy(g1, acc.at[idx_d.at[CH2 - 1]], sems1).wait()

    half(0)
    half(1)
    plsc.subcore_barrier()
    pltpu.sync_copy(acc.at[pl.ds(s * RPT, RPT)],
                    out_h.at[c, pl.ds(s * RPT, RPT)])


@functools.lru_cache(maxsize=None)
def _agg_call():
    return pl.kernel(
        _agg_body,
        out_type=jax.ShapeDtypeStruct((NC, NP, D), jnp.float32),
        mesh=_mesh(),
        scratch_types=[
            pltpu.VMEM((CH2, K), jnp.int32),
            pltpu.VMEM((CH2, K), jnp.int32),
            pltpu.VMEM((128, D), jnp.float32),
            pltpu.VMEM((128, D), jnp.float32),
            pltpu.VMEM_SHARED((NP, D), jnp.float32),
            pltpu.SemaphoreType.DMA,
            pltpu.SemaphoreType.DMA,
            pltpu.SemaphoreType.DMA,
            pltpu.SemaphoreType.DMA,
        ],
    )


def _norm_body(dsr, ddr, x, ns_o, nd_o, xs_o):
    ones = jnp.ones((NW, 1), jnp.float32)
    dn = (((0,), (0,)), ((), ()))
    deg_s = lax.dot_general(dsr[...], ones, dn, preferred_element_type=jnp.float32)
    deg_d = lax.dot_general(ddr[...], ones, dn, preferred_element_type=jnp.float32)
    n_s = lax.rsqrt(jnp.maximum(deg_s, 1.0))
    n_d = lax.rsqrt(jnp.maximum(deg_d, 1.0))
    ns_o[...] = n_s
    nd_o[...] = n_d
    xs_o[...] = x[...] * n_s


def _norm_call(degs, degd, x):
    deg_spec = pl.BlockSpec((NW, BN), lambda i: (0, i))
    n_spec = pl.BlockSpec((BN, 1), lambda i: (i, 0))
    return pl.pallas_call(
        _norm_body,
        grid=(NB,),
        in_specs=[deg_spec, deg_spec,
                  pl.BlockSpec((BN, D), lambda i: (i, 0))],
        out_specs=[n_spec, n_spec, pl.BlockSpec((BN, D), lambda i: (i, 0))],
        out_shape=[jax.ShapeDtypeStruct((NP, 1), jnp.float32),
                   jax.ShapeDtypeStruct((NP, 1), jnp.float32),
                   jax.ShapeDtypeStruct((NP, D), jnp.float32)],
    )(degs, degd, x)


_P0_SPEC = pl.BlockSpec((1, BN, D), lambda i: (0, i, 0))
_P1_SPEC = pl.BlockSpec((1, BN, D), lambda i: (1, i, 0))


def _layer_body(p0, p1, nd, ns, w, b, o):
    agg = (p0[0] + p1[0]) * nd[...]
    h = jnp.dot(agg, w[...], preferred_element_type=jnp.float32) + b[...]
    o[...] = jnp.maximum(h, 0.0) * ns[...]


def _layer_call(p, nd, ns, w, b):
    blk = pl.BlockSpec((BN, D), lambda i: (i, 0))
    n_spec = pl.BlockSpec((BN, 1), lambda i: (i, 0))
    return pl.pallas_call(
        _layer_body,
        grid=(NB,),
        in_specs=[_P0_SPEC, _P1_SPEC, n_spec, n_spec,
                  pl.BlockSpec((D, D), lambda i: (0, 0)),
                  pl.BlockSpec((1, D), lambda i: (0, 0))],
        out_specs=blk,
        out_shape=jax.ShapeDtypeStruct((NP, D), jnp.float32),
    )(p, p, nd, ns, w, b)


def _head_body(p0, p1, nd, w, b, wp, bp, o, acc):
    i = pl.program_id(0)
    agg = (p0[0] + p1[0]) * nd[...]
    h = jnp.dot(agg, w[...], preferred_element_type=jnp.float32) + b[...]
    h = jnp.maximum(h, 0.0)
    row = lax.broadcasted_iota(jnp.int32, (BN, 1), 0) + i * BN
    h = jnp.where(row < N, h, 0.0)
    colsum = jnp.sum(h, axis=0, keepdims=True)

    @pl.when(i == 0)
    def _():
        acc[...] = colsum

    @pl.when(i > 0)
    def _():
        acc[...] = acc[...] + colsum

    @pl.when(i == NB - 1)
    def _():
        hg = acc[...] * (1.0 / N)
        o[...] = jnp.dot(hg, wp[...], preferred_element_type=jnp.float32) + bp[...]


def _head_call(p, nd, w, b, wpT, bp):
    return pl.pallas_call(
        _head_body,
        grid=(NB,),
        in_specs=[_P0_SPEC, _P1_SPEC, pl.BlockSpec((BN, 1), lambda i: (i, 0)),
                  pl.BlockSpec((D, D), lambda i: (0, 0)),
                  pl.BlockSpec((1, D), lambda i: (0, 0)),
                  pl.BlockSpec((D, C), lambda i: (0, 0)),
                  pl.BlockSpec((1, C), lambda i: (0, 0))],
        out_specs=pl.BlockSpec((1, C), lambda i: (0, 0)),
        out_shape=jax.ShapeDtypeStruct((1, C), jnp.float32),
        scratch_shapes=[pltpu.VMEM((1, D), jnp.float32)],
    )(p, p, nd, w, b, wpT, bp)


def kernel(features, edge_index, W1, b1, W2, b2, W3, b3, Wp, bp):
    src3 = edge_index[0].reshape(NW, CH, K)
    dst3 = edge_index[1].reshape(NW, CH, K)
    xpad = jnp.pad(features, ((0, NP - N), (0, 0)))

    degs, degd = _deg_call()(edge_index[0], edge_index[1])
    norm_src, norm_dst, h = _norm_call(degs.reshape(NW, NP),
                                       degd.reshape(NW, NP), xpad)
    for w, b in ((W1, b1), (W2, b2)):
        p = _agg_call()(h, src3, dst3)
        h = _layer_call(p, norm_dst, norm_src, w, b.reshape(1, D))
    p = _agg_call()(h, src3, dst3)
    return _head_call(p, norm_dst, W3, b3.reshape(1, D),
                      Wp.T, bp.reshape(1, C))


# speedup vs baseline: 1.0221x; 1.0221x over previous
import functools

import jax
import jax.numpy as jnp
from jax import lax
from jax.experimental import pallas as pl
from jax.experimental.pallas import tpu as pltpu
from jax.experimental.pallas import tpu_sc as plsc

N = 10000
E = 320000
D = 128
C = 50
NC = 2
NS = 16
NW = NC * NS
EPW = E // NW
K = 125
CH = EPW // K
CH2 = CH // 2
NP = 10240
RPT = NP // NS
NB = 10
BN = NP // NB

@functools.lru_cache(maxsize=None)
def _mesh():
    return plsc.VectorSubcoreMesh(core_axis_name="c", subcore_axis_name="s")


def _deg_body(src_i, dst_i, outs_h, outd_h, idx_s, idx_d, hist_s, hist_d):
    c = lax.axis_index("c")
    s = lax.axis_index("s")
    wid = c * NS + s
    pltpu.sync_copy(src_i.at[pl.ds(wid * EPW, EPW)], idx_s)
    pltpu.sync_copy(dst_i.at[pl.ds(wid * EPW, EPW)], idx_d)

    zeros16 = jnp.zeros((16,), jnp.float32)

    def z(i, carry):
        hist_s[pl.ds(i * 16, 16)] = zeros16
        hist_d[pl.ds(i * 16, 16)] = zeros16
        return carry

    lax.fori_loop(0, NP // 16, z, 0)

    ones16 = jnp.ones((16,), jnp.float32)

    def acc(i, carry):
        plsc.addupdate_scatter(hist_s, [idx_s[pl.ds(i * 16, 16)]], ones16)
        plsc.addupdate_scatter(hist_d, [idx_d[pl.ds(i * 16, 16)]], ones16)
        return carry

    lax.fori_loop(0, EPW // 16, acc, 0)
    pltpu.sync_copy(hist_s, outs_h.at[pl.ds(wid * NP, NP)])
    pltpu.sync_copy(hist_d, outd_h.at[pl.ds(wid * NP, NP)])


@functools.lru_cache(maxsize=None)
def _deg_call():
    return pl.kernel(
        _deg_body,
        out_type=(jax.ShapeDtypeStruct((NW * NP,), jnp.float32),
                  jax.ShapeDtypeStruct((NW * NP,), jnp.float32)),
        mesh=_mesh(),
        scratch_types=[
            pltpu.VMEM((EPW,), jnp.int32),
            pltpu.VMEM((EPW,), jnp.int32),
            pltpu.VMEM((NP,), jnp.float32),
            pltpu.VMEM((NP,), jnp.float32),
        ],
        compiler_params=pltpu.CompilerParams(needs_layout_passes=False),
    )


def _agg_body(h_h, src_i, dst_i, out_h, idx_s, idx_d, rows0, rows1,
              acc, semg0, semg1, sems0, sems1):
    c = lax.axis_index("c")
    s = lax.axis_index("s")
    wid = c * NS + s

    zf = jnp.zeros((16,), jnp.float32)

    def zz(i, carry):
        for cc in range(8):
            rows0[i, pl.ds(cc * 16, 16)] = zf
        return carry

    lax.fori_loop(0, 128, zz, 0)
    for t in range(RPT // 128):
        pltpu.sync_copy(rows0, acc.at[pl.ds(s * RPT + t * 128, 128)])
    plsc.subcore_barrier()

    g0 = rows0.at[pl.ds(0, K)]
    g1 = rows1.at[pl.ds(0, K)]

    def half(hf):
        pltpu.sync_copy(src_i.at[wid, pl.ds(hf * CH2, CH2)], idx_s)
        pltpu.sync_copy(dst_i.at[wid, pl.ds(hf * CH2, CH2)], idx_d)
        pltpu.async_copy(h_h.at[idx_s.at[0]], g0, semg0)

        def pair(i, carry):
            j = 2 * i
            pltpu.make_async_copy(h_h.at[idx_s.at[j]], g0, semg0).wait()
            pltpu.make_async_copy(g0, acc.at[idx_d.at[j]], sems0).start(add=True)

            @pl.when(i > 0)
            def _():
                pltpu.make_async_copy(g1, acc.at[idx_d.at[j - 1]], sems1).wait()

            pltpu.async_copy(h_h.at[idx_s.at[j + 1]], g1, semg1)
            pltpu.make_async_copy(h_h.at[idx_s.at[j + 1]], g1, semg1).wait()
            pltpu.make_async_copy(g1, acc.at[idx_d.at[j + 1]], sems1).start(add=True)
            pltpu.make_async_copy(g0, acc.at[idx_d.at[j]], sems0).wait()

            @pl.when(j + 2 < CH2)
            def _():
                pltpu.async_copy(h_h.at[idx_s.at[j + 2]], g0, semg0)

            return carry

        lax.fori_loop(0, CH2 // 2, pair, 0)
        pltpu.make_async_copy(g1, acc.at[idx_d.at[CH2 - 1]], sems1).wait()

    half(0)
    half(1)
    plsc.subcore_barrier()
    pltpu.sync_copy(acc.at[pl.ds(s * RPT, RPT)],
                    out_h.at[c, pl.ds(s * RPT, RPT)])


@functools.lru_cache(maxsize=None)
def _agg_call():
    return pl.kernel(
        _agg_body,
        out_type=jax.ShapeDtypeStruct((NC, NP, D), jnp.float32),
        mesh=_mesh(),
        scratch_types=[
            pltpu.VMEM((CH2, K), jnp.int32),
            pltpu.VMEM((CH2, K), jnp.int32),
            pltpu.VMEM((128, D), jnp.float32),
            pltpu.VMEM((128, D), jnp.float32),
            pltpu.VMEM_SHARED((NP, D), jnp.float32),
            pltpu.SemaphoreType.DMA,
            pltpu.SemaphoreType.DMA,
            pltpu.SemaphoreType.DMA,
            pltpu.SemaphoreType.DMA,
        ],
    )


def _norm_body(dsr, ddr, x, ns_o, nd_o, xs_o):
    ones = jnp.ones((NW, 1), jnp.float32)
    dn = (((0,), (0,)), ((), ()))
    deg_s = lax.dot_general(dsr[...], ones, dn, preferred_element_type=jnp.float32)
    deg_d = lax.dot_general(ddr[...], ones, dn, preferred_element_type=jnp.float32)
    n_s = lax.rsqrt(jnp.maximum(deg_s, 1.0))
    n_d = lax.rsqrt(jnp.maximum(deg_d, 1.0))
    ns_o[...] = n_s
    nd_o[...] = n_d
    xs_o[...] = x[...] * n_s


def _norm_call(degs, degd, x):
    deg_spec = pl.BlockSpec((NW, BN), lambda i: (0, i))
    n_spec = pl.BlockSpec((BN, 1), lambda i: (i, 0))
    return pl.pallas_call(
        _norm_body,
        grid=(NB,),
        in_specs=[deg_spec, deg_spec,
                  pl.BlockSpec((BN, D), lambda i: (i, 0))],
        out_specs=[n_spec, n_spec, pl.BlockSpec((BN, D), lambda i: (i, 0))],
        out_shape=[jax.ShapeDtypeStruct((NP, 1), jnp.float32),
                   jax.ShapeDtypeStruct((NP, 1), jnp.float32),
                   jax.ShapeDtypeStruct((NP, D), jnp.float32)],
    )(degs, degd, x)


_P0_SPEC = pl.BlockSpec((1, BN, D), lambda i: (0, i, 0))
_P1_SPEC = pl.BlockSpec((1, BN, D), lambda i: (1, i, 0))


def _layer_body(p0, p1, nd, ns, w, b, o):
    agg = (p0[0] + p1[0]) * nd[...]
    h = jnp.dot(agg, w[...], preferred_element_type=jnp.float32) + b[...]
    o[...] = jnp.maximum(h, 0.0) * ns[...]


def _layer_call(p, nd, ns, w, b):
    blk = pl.BlockSpec((BN, D), lambda i: (i, 0))
    n_spec = pl.BlockSpec((BN, 1), lambda i: (i, 0))
    return pl.pallas_call(
        _layer_body,
        grid=(NB,),
        in_specs=[_P0_SPEC, _P1_SPEC, n_spec, n_spec,
                  pl.BlockSpec((D, D), lambda i: (0, 0)),
                  pl.BlockSpec((1, D), lambda i: (0, 0))],
        out_specs=blk,
        out_shape=jax.ShapeDtypeStruct((NP, D), jnp.float32),
    )(p, p, nd, ns, w, b)


def _head_body(p0, p1, nd, w, b, wp, bp, o, acc):
    i = pl.program_id(0)
    agg = (p0[0] + p1[0]) * nd[...]
    h = jnp.dot(agg, w[...], preferred_element_type=jnp.float32) + b[...]
    h = jnp.maximum(h, 0.0)
    row = lax.broadcasted_iota(jnp.int32, (BN, 1), 0) + i * BN
    h = jnp.where(row < N, h, 0.0)
    colsum = jnp.sum(h, axis=0, keepdims=True)

    @pl.when(i == 0)
    def _():
        acc[...] = colsum

    @pl.when(i > 0)
    def _():
        acc[...] = acc[...] + colsum

    @pl.when(i == NB - 1)
    def _():
        hg = acc[...] * (1.0 / N)
        o[...] = jnp.dot(hg, wp[...], preferred_element_type=jnp.float32) + bp[...]


def _head_call(p, nd, w, b, wpT, bp):
    return pl.pallas_call(
        _head_body,
        grid=(NB,),
        in_specs=[_P0_SPEC, _P1_SPEC, pl.BlockSpec((BN, 1), lambda i: (i, 0)),
                  pl.BlockSpec((D, D), lambda i: (0, 0)),
                  pl.BlockSpec((1, D), lambda i: (0, 0)),
                  pl.BlockSpec((D, C), lambda i: (0, 0)),
                  pl.BlockSpec((1, C), lambda i: (0, 0))],
        out_specs=pl.BlockSpec((1, C), lambda i: (0, 0)),
        out_shape=jax.ShapeDtypeStruct((1, C), jnp.float32),
        scratch_shapes=[pltpu.VMEM((1, D), jnp.float32)],
    )(p, p, nd, w, b, wpT, bp)


def kernel(features, edge_index, W1, b1, W2, b2, W3, b3, Wp, bp):
    src3 = edge_index[0].reshape(NW, CH, K)
    dst3 = edge_index[1].reshape(NW, CH, K)
    xpad = jnp.pad(features, ((0, NP - N), (0, 0)))

    degs, degd = _deg_call()(edge_index[0], edge_index[1])
    norm_src, norm_dst, h = _norm_call(degs.reshape(NW, NP),
                                       degd.reshape(NW, NP), xpad)
    for w, b in ((W1, b1), (W2, b2)):
        p = _agg_call()(h, src3, dst3)
        h = _layer_call(p, norm_dst, norm_src, w, b.reshape(1, D))
    p = _agg_call()(h, src3, dst3)
    return _head_call(p, norm_dst, W3, b3.reshape(1, D),
                      Wp.T, bp.reshape(1, C))


# speedup vs baseline: 1.1822x; 1.1566x over previous
import functools

import jax
import jax.numpy as jnp
from jax import lax
from jax.experimental import pallas as pl
from jax.experimental.pallas import tpu as pltpu
from jax.experimental.pallas import tpu_sc as plsc

N = 10000
E = 320000
D = 128
C = 50
NC = 2
NS = 16
NW = NC * NS
EPW = E // NW
K = 125
CH = EPW // K
CH2 = CH // 2
NP = 10240
RPT = NP // NS
NB = 10
BN = NP // NB

@functools.lru_cache(maxsize=None)
def _mesh():
    return plsc.VectorSubcoreMesh(core_axis_name="c", subcore_axis_name="s")


def _deg_body(src_i, dst_i, outs_h, outd_h, idx_s, idx_d, hist_s, hist_d):
    c = lax.axis_index("c")
    s = lax.axis_index("s")
    wid = c * NS + s
    pltpu.sync_copy(src_i.at[pl.ds(wid * EPW, EPW)], idx_s)
    pltpu.sync_copy(dst_i.at[pl.ds(wid * EPW, EPW)], idx_d)

    zeros16 = jnp.zeros((16,), jnp.float32)

    def z(i, carry):
        hist_s[pl.ds(i * 16, 16)] = zeros16
        hist_d[pl.ds(i * 16, 16)] = zeros16
        return carry

    lax.fori_loop(0, NP // 16, z, 0)

    ones16 = jnp.ones((16,), jnp.float32)

    def acc(i, carry):
        plsc.addupdate_scatter(hist_s, [idx_s[pl.ds(i * 16, 16)]], ones16)
        plsc.addupdate_scatter(hist_d, [idx_d[pl.ds(i * 16, 16)]], ones16)
        return carry

    lax.fori_loop(0, EPW // 16, acc, 0)
    pltpu.sync_copy(hist_s, outs_h.at[pl.ds(wid * NP, NP)])
    pltpu.sync_copy(hist_d, outd_h.at[pl.ds(wid * NP, NP)])


@functools.lru_cache(maxsize=None)
def _deg_call():
    return pl.kernel(
        _deg_body,
        out_type=(jax.ShapeDtypeStruct((NW * NP,), jnp.float32),
                  jax.ShapeDtypeStruct((NW * NP,), jnp.float32)),
        mesh=_mesh(),
        scratch_types=[
            pltpu.VMEM((EPW,), jnp.int32),
            pltpu.VMEM((EPW,), jnp.int32),
            pltpu.VMEM((NP,), jnp.float32),
            pltpu.VMEM((NP,), jnp.float32),
        ],
        compiler_params=pltpu.CompilerParams(needs_layout_passes=False),
    )


def _agg_body(h_h, src_i, dst_i, out_h, idx_s, idx_d, rows0, rows1,
              acc, semg0, semg1, sems0, sems1):
    c = lax.axis_index("c")
    s = lax.axis_index("s")
    wid = c * NS + s

    zf = jnp.zeros((16,), jnp.float32)

    def zz(i, carry):
        for cc in range(8):
            rows0[i, pl.ds(cc * 16, 16)] = zf
        return carry

    lax.fori_loop(0, 128, zz, 0)
    for t in range(RPT // 128):
        pltpu.sync_copy(rows0, acc.at[pl.ds(s * RPT + t * 128, 128)])
    plsc.subcore_barrier()

    g0 = rows0.at[pl.ds(0, K)]
    g1 = rows1.at[pl.ds(0, K)]

    def half(hf):
        pltpu.sync_copy(src_i.at[wid, pl.ds(hf * CH2, CH2)], idx_s)
        pltpu.sync_copy(dst_i.at[wid, pl.ds(hf * CH2, CH2)], idx_d)
        pltpu.async_copy(h_h.at[idx_s.at[0]], g0, semg0)

        def pair(i, carry):
            j = 2 * i
            pltpu.async_copy(h_h.at[idx_s.at[j + 1]], g1, semg1)
            pltpu.make_async_copy(h_h.at[idx_s.at[j]], g0, semg0).wait()
            pltpu.sync_copy(g0, acc.at[idx_d.at[j]], add=True)

            @pl.when(j + 2 < CH2)
            def _():
                pltpu.async_copy(h_h.at[idx_s.at[j + 2]], g0, semg0)

            pltpu.make_async_copy(h_h.at[idx_s.at[j + 1]], g1, semg1).wait()
            pltpu.sync_copy(g1, acc.at[idx_d.at[j + 1]], add=True)
            return carry

        lax.fori_loop(0, CH2 // 2, pair, 0)

    half(0)
    half(1)
    plsc.subcore_barrier()
    pltpu.sync_copy(acc.at[pl.ds(s * RPT, RPT)],
                    out_h.at[c, pl.ds(s * RPT, RPT)])


@functools.lru_cache(maxsize=None)
def _agg_call():
    return pl.kernel(
        _agg_body,
        out_type=jax.ShapeDtypeStruct((NC, NP, D), jnp.float32),
        mesh=_mesh(),
        scratch_types=[
            pltpu.VMEM((CH2, K), jnp.int32),
            pltpu.VMEM((CH2, K), jnp.int32),
            pltpu.VMEM((128, D), jnp.float32),
            pltpu.VMEM((128, D), jnp.float32),
            pltpu.VMEM_SHARED((NP, D), jnp.float32),
            pltpu.SemaphoreType.DMA,
            pltpu.SemaphoreType.DMA,
            pltpu.SemaphoreType.DMA,
            pltpu.SemaphoreType.DMA,
        ],
    )


def _norm_body(dsr, ddr, x, ns_o, nd_o, xs_o):
    ones = jnp.ones((NW, 1), jnp.float32)
    dn = (((0,), (0,)), ((), ()))
    deg_s = lax.dot_general(dsr[...], ones, dn, preferred_element_type=jnp.float32)
    deg_d = lax.dot_general(ddr[...], ones, dn, preferred_element_type=jnp.float32)
    n_s = lax.rsqrt(jnp.maximum(deg_s, 1.0))
    n_d = lax.rsqrt(jnp.maximum(deg_d, 1.0))
    ns_o[...] = n_s
    nd_o[...] = n_d
    xs_o[...] = x[...] * n_s


def _norm_call(degs, degd, x):
    deg_spec = pl.BlockSpec((NW, BN), lambda i: (0, i))
    n_spec = pl.BlockSpec((BN, 1), lambda i: (i, 0))
    return pl.pallas_call(
        _norm_body,
        grid=(NB,),
        in_specs=[deg_spec, deg_spec,
                  pl.BlockSpec((BN, D), lambda i: (i, 0))],
        out_specs=[n_spec, n_spec, pl.BlockSpec((BN, D), lambda i: (i, 0))],
        out_shape=[jax.ShapeDtypeStruct((NP, 1), jnp.float32),
                   jax.ShapeDtypeStruct((NP, 1), jnp.float32),
                   jax.ShapeDtypeStruct((NP, D), jnp.float32)],
    )(degs, degd, x)


_P0_SPEC = pl.BlockSpec((1, BN, D), lambda i: (0, i, 0))
_P1_SPEC = pl.BlockSpec((1, BN, D), lambda i: (1, i, 0))


def _layer_body(p0, p1, nd, ns, w, b, o):
    agg = (p0[0] + p1[0]) * nd[...]
    h = jnp.dot(agg, w[...], preferred_element_type=jnp.float32) + b[...]
    o[...] = jnp.maximum(h, 0.0) * ns[...]


def _layer_call(p, nd, ns, w, b):
    blk = pl.BlockSpec((BN, D), lambda i: (i, 0))
    n_spec = pl.BlockSpec((BN, 1), lambda i: (i, 0))
    return pl.pallas_call(
        _layer_body,
        grid=(NB,),
        in_specs=[_P0_SPEC, _P1_SPEC, n_spec, n_spec,
                  pl.BlockSpec((D, D), lambda i: (0, 0)),
                  pl.BlockSpec((1, D), lambda i: (0, 0))],
        out_specs=blk,
        out_shape=jax.ShapeDtypeStruct((NP, D), jnp.float32),
    )(p, p, nd, ns, w, b)


def _head_body(p0, p1, nd, w, b, wp, bp, o, acc):
    i = pl.program_id(0)
    agg = (p0[0] + p1[0]) * nd[...]
    h = jnp.dot(agg, w[...], preferred_element_type=jnp.float32) + b[...]
    h = jnp.maximum(h, 0.0)
    row = lax.broadcasted_iota(jnp.int32, (BN, 1), 0) + i * BN
    h = jnp.where(row < N, h, 0.0)
    colsum = jnp.sum(h, axis=0, keepdims=True)

    @pl.when(i == 0)
    def _():
        acc[...] = colsum

    @pl.when(i > 0)
    def _():
        acc[...] = acc[...] + colsum

    @pl.when(i == NB - 1)
    def _():
        hg = acc[...] * (1.0 / N)
        o[...] = jnp.dot(hg, wp[...], preferred_element_type=jnp.float32) + bp[...]


def _head_call(p, nd, w, b, wpT, bp):
    return pl.pallas_call(
        _head_body,
        grid=(NB,),
        in_specs=[_P0_SPEC, _P1_SPEC, pl.BlockSpec((BN, 1), lambda i: (i, 0)),
                  pl.BlockSpec((D, D), lambda i: (0, 0)),
                  pl.BlockSpec((1, D), lambda i: (0, 0)),
                  pl.BlockSpec((D, C), lambda i: (0, 0)),
                  pl.BlockSpec((1, C), lambda i: (0, 0))],
        out_specs=pl.BlockSpec((1, C), lambda i: (0, 0)),
        out_shape=jax.ShapeDtypeStruct((1, C), jnp.float32),
        scratch_shapes=[pltpu.VMEM((1, D), jnp.float32)],
    )(p, p, nd, w, b, wpT, bp)


def kernel(features, edge_index, W1, b1, W2, b2, W3, b3, Wp, bp):
    src3 = edge_index[0].reshape(NW, CH, K)
    dst3 = edge_index[1].reshape(NW, CH, K)
    xpad = jnp.pad(features, ((0, NP - N), (0, 0)))

    degs, degd = _deg_call()(edge_index[0], edge_index[1])
    norm_src, norm_dst, h = _norm_call(degs.reshape(NW, NP),
                                       degd.reshape(NW, NP), xpad)
    for w, b in ((W1, b1), (W2, b2)):
        p = _agg_call()(h, src3, dst3)
        h = _layer_call(p, norm_dst, norm_src, w, b.reshape(1, D))
    p = _agg_call()(h, src3, dst3)
    return _head_call(p, norm_dst, W3, b3.reshape(1, D),
                      Wp.T, bp.reshape(1, C))


# speedup vs baseline: 1.2099x; 1.0234x over previous
import functools

import jax
import jax.numpy as jnp
from jax import lax
from jax.experimental import pallas as pl
from jax.experimental.pallas import tpu as pltpu
from jax.experimental.pallas import tpu_sc as plsc

N = 10000
E = 320000
D = 128
C = 50
NC = 2
NS = 16
NW = NC * NS
EPW = E // NW
K = 125
CH = EPW // K
CH2 = CH // 2
NP = 10240
RPT = NP // NS
NB = 5
BN = NP // NB

@functools.lru_cache(maxsize=None)
def _mesh():
    return plsc.VectorSubcoreMesh(core_axis_name="c", subcore_axis_name="s")


def _deg_body(src_i, dst_i, outs_h, outd_h, idx_s, idx_d, hist_s, hist_d):
    c = lax.axis_index("c")
    s = lax.axis_index("s")
    wid = c * NS + s
    pltpu.sync_copy(src_i.at[pl.ds(wid * EPW, EPW)], idx_s)
    pltpu.sync_copy(dst_i.at[pl.ds(wid * EPW, EPW)], idx_d)

    zeros16 = jnp.zeros((16,), jnp.float32)

    def z(i, carry):
        hist_s[pl.ds(i * 16, 16)] = zeros16
        hist_d[pl.ds(i * 16, 16)] = zeros16
        return carry

    lax.fori_loop(0, NP // 16, z, 0)

    ones16 = jnp.ones((16,), jnp.float32)

    def acc(i, carry):
        plsc.addupdate_scatter(hist_s, [idx_s[pl.ds(i * 16, 16)]], ones16)
        plsc.addupdate_scatter(hist_d, [idx_d[pl.ds(i * 16, 16)]], ones16)
        return carry

    lax.fori_loop(0, EPW // 16, acc, 0)
    pltpu.sync_copy(hist_s, outs_h.at[pl.ds(wid * NP, NP)])
    pltpu.sync_copy(hist_d, outd_h.at[pl.ds(wid * NP, NP)])


@functools.lru_cache(maxsize=None)
def _deg_call():
    return pl.kernel(
        _deg_body,
        out_type=(jax.ShapeDtypeStruct((NW * NP,), jnp.float32),
                  jax.ShapeDtypeStruct((NW * NP,), jnp.float32)),
        mesh=_mesh(),
        scratch_types=[
            pltpu.VMEM((EPW,), jnp.int32),
            pltpu.VMEM((EPW,), jnp.int32),
            pltpu.VMEM((NP,), jnp.float32),
            pltpu.VMEM((NP,), jnp.float32),
        ],
        compiler_params=pltpu.CompilerParams(needs_layout_passes=False),
    )


def _agg_body(h_h, src_i, dst_i, out_h, idx_s, idx_d, rows0, rows1,
              acc, semg0, semg1, sems0, sems1):
    c = lax.axis_index("c")
    s = lax.axis_index("s")
    wid = c * NS + s

    zf = jnp.zeros((16,), jnp.float32)

    def zz(i, carry):
        for cc in range(8):
            rows0[i, pl.ds(cc * 16, 16)] = zf
        return carry

    lax.fori_loop(0, 128, zz, 0)
    for t in range(RPT // 128):
        pltpu.sync_copy(rows0, acc.at[pl.ds(s * RPT + t * 128, 128)])
    plsc.subcore_barrier()

    g0 = rows0.at[pl.ds(0, K)]
    g1 = rows1.at[pl.ds(0, K)]

    def half(hf):
        pltpu.sync_copy(src_i.at[wid, pl.ds(hf * CH2, CH2)], idx_s)
        pltpu.sync_copy(dst_i.at[wid, pl.ds(hf * CH2, CH2)], idx_d)
        pltpu.async_copy(h_h.at[idx_s.at[0]], g0, semg0)

        def pair(i, carry):
            j = 2 * i
            pltpu.async_copy(h_h.at[idx_s.at[j + 1]], g1, semg1)
            pltpu.make_async_copy(h_h.at[idx_s.at[j]], g0, semg0).wait()
            pltpu.sync_copy(g0, acc.at[idx_d.at[j]], add=True)

            @pl.when(j + 2 < CH2)
            def _():
                pltpu.async_copy(h_h.at[idx_s.at[j + 2]], g0, semg0)

            pltpu.make_async_copy(h_h.at[idx_s.at[j + 1]], g1, semg1).wait()
            pltpu.sync_copy(g1, acc.at[idx_d.at[j + 1]], add=True)
            return carry

        lax.fori_loop(0, CH2 // 2, pair, 0)

    half(0)
    half(1)
    plsc.subcore_barrier()
    pltpu.sync_copy(acc.at[pl.ds(s * RPT, RPT)],
                    out_h.at[c, pl.ds(s * RPT, RPT)])


@functools.lru_cache(maxsize=None)
def _agg_call():
    return pl.kernel(
        _agg_body,
        out_type=jax.ShapeDtypeStruct((NC, NP, D), jnp.float32),
        mesh=_mesh(),
        scratch_types=[
            pltpu.VMEM((CH2, K), jnp.int32),
            pltpu.VMEM((CH2, K), jnp.int32),
            pltpu.VMEM((128, D), jnp.float32),
            pltpu.VMEM((128, D), jnp.float32),
            pltpu.VMEM_SHARED((NP, D), jnp.float32),
            pltpu.SemaphoreType.DMA,
            pltpu.SemaphoreType.DMA,
            pltpu.SemaphoreType.DMA,
            pltpu.SemaphoreType.DMA,
        ],
    )


def _norm_body(dsr, ddr, x, ns_o, nd_o, xs_o):
    ones = jnp.ones((NW, 1), jnp.float32)
    dn = (((0,), (0,)), ((), ()))
    deg_s = lax.dot_general(dsr[...], ones, dn, preferred_element_type=jnp.float32)
    deg_d = lax.dot_general(ddr[...], ones, dn, preferred_element_type=jnp.float32)
    n_s = lax.rsqrt(jnp.maximum(deg_s, 1.0))
    n_d = lax.rsqrt(jnp.maximum(deg_d, 1.0))
    ns_o[...] = n_s
    nd_o[...] = n_d
    xs_o[...] = x[...] * n_s


def _norm_call(degs, degd, x):
    deg_spec = pl.BlockSpec((NW, BN), lambda i: (0, i))
    n_spec = pl.BlockSpec((BN, 1), lambda i: (i, 0))
    return pl.pallas_call(
        _norm_body,
        grid=(NB,),
        in_specs=[deg_spec, deg_spec,
                  pl.BlockSpec((BN, D), lambda i: (i, 0))],
        out_specs=[n_spec, n_spec, pl.BlockSpec((BN, D), lambda i: (i, 0))],
        out_shape=[jax.ShapeDtypeStruct((NP, 1), jnp.float32),
                   jax.ShapeDtypeStruct((NP, 1), jnp.float32),
                   jax.ShapeDtypeStruct((NP, D), jnp.float32)],
    )(degs, degd, x)


_P0_SPEC = pl.BlockSpec((1, BN, D), lambda i: (0, i, 0))
_P1_SPEC = pl.BlockSpec((1, BN, D), lambda i: (1, i, 0))


def _layer_body(p0, p1, nd, ns, w, b, o):
    agg = (p0[0] + p1[0]) * nd[...]
    h = jnp.dot(agg, w[...], preferred_element_type=jnp.float32) + b[...]
    o[...] = jnp.maximum(h, 0.0) * ns[...]


def _layer_call(p, nd, ns, w, b):
    blk = pl.BlockSpec((BN, D), lambda i: (i, 0))
    n_spec = pl.BlockSpec((BN, 1), lambda i: (i, 0))
    return pl.pallas_call(
        _layer_body,
        grid=(NB,),
        in_specs=[_P0_SPEC, _P1_SPEC, n_spec, n_spec,
                  pl.BlockSpec((D, D), lambda i: (0, 0)),
                  pl.BlockSpec((1, D), lambda i: (0, 0))],
        out_specs=blk,
        out_shape=jax.ShapeDtypeStruct((NP, D), jnp.float32),
    )(p, p, nd, ns, w, b)


def _head_body(p0, p1, nd, w, b, wp, bp, o, acc):
    i = pl.program_id(0)
    agg = (p0[0] + p1[0]) * nd[...]
    h = jnp.dot(agg, w[...], preferred_element_type=jnp.float32) + b[...]
    h = jnp.maximum(h, 0.0)
    row = lax.broadcasted_iota(jnp.int32, (BN, 1), 0) + i * BN
    h = jnp.where(row < N, h, 0.0)
    colsum = jnp.sum(h, axis=0, keepdims=True)

    @pl.when(i == 0)
    def _():
        acc[...] = colsum

    @pl.when(i > 0)
    def _():
        acc[...] = acc[...] + colsum

    @pl.when(i == NB - 1)
    def _():
        hg = acc[...] * (1.0 / N)
        o[...] = jnp.dot(hg, wp[...], preferred_element_type=jnp.float32) + bp[...]


def _head_call(p, nd, w, b, wpT, bp):
    return pl.pallas_call(
        _head_body,
        grid=(NB,),
        in_specs=[_P0_SPEC, _P1_SPEC, pl.BlockSpec((BN, 1), lambda i: (i, 0)),
                  pl.BlockSpec((D, D), lambda i: (0, 0)),
                  pl.BlockSpec((1, D), lambda i: (0, 0)),
                  pl.BlockSpec((D, C), lambda i: (0, 0)),
                  pl.BlockSpec((1, C), lambda i: (0, 0))],
        out_specs=pl.BlockSpec((1, C), lambda i: (0, 0)),
        out_shape=jax.ShapeDtypeStruct((1, C), jnp.float32),
        scratch_shapes=[pltpu.VMEM((1, D), jnp.float32)],
    )(p, p, nd, w, b, wpT, bp)


def kernel(features, edge_index, W1, b1, W2, b2, W3, b3, Wp, bp):
    src3 = edge_index[0].reshape(NW, CH, K)
    dst3 = edge_index[1].reshape(NW, CH, K)
    xpad = jnp.pad(features, ((0, NP - N), (0, 0)))

    degs, degd = _deg_call()(edge_index[0], edge_index[1])
    norm_src, norm_dst, h = _norm_call(degs.reshape(NW, NP),
                                       degd.reshape(NW, NP), xpad)
    for w, b in ((W1, b1), (W2, b2)):
        p = _agg_call()(h, src3, dst3)
        h = _layer_call(p, norm_dst, norm_src, w, b.reshape(1, D))
    p = _agg_call()(h, src3, dst3)
    return _head_call(p, norm_dst, W3, b3.reshape(1, D),
                      Wp.T, bp.reshape(1, C))


# speedup vs baseline: 1.2207x; 1.0089x over previous
import functools

import jax
import jax.numpy as jnp
from jax import lax
from jax.experimental import pallas as pl
from jax.experimental.pallas import tpu as pltpu
from jax.experimental.pallas import tpu_sc as plsc

N = 10000
E = 320000
D = 128
C = 50
NC = 2
NS = 16
NW = NC * NS
EPW = E // NW
K = 125
CH = EPW // K
CH2 = CH // 2
NP = 10240
RPT = NP // NS
NB = 4
BN = NP // NB

@functools.lru_cache(maxsize=None)
def _mesh():
    return plsc.VectorSubcoreMesh(core_axis_name="c", subcore_axis_name="s")


def _deg_body(src_i, dst_i, outs_h, outd_h, idx_s, idx_d, hist_s, hist_d):
    c = lax.axis_index("c")
    s = lax.axis_index("s")
    wid = c * NS + s
    pltpu.sync_copy(src_i.at[pl.ds(wid * EPW, EPW)], idx_s)
    pltpu.sync_copy(dst_i.at[pl.ds(wid * EPW, EPW)], idx_d)

    zeros16 = jnp.zeros((16,), jnp.float32)

    def z(i, carry):
        hist_s[pl.ds(i * 16, 16)] = zeros16
        hist_d[pl.ds(i * 16, 16)] = zeros16
        return carry

    lax.fori_loop(0, NP // 16, z, 0)

    ones16 = jnp.ones((16,), jnp.float32)

    def acc(i, carry):
        for u in range(4):
            t = i * 4 + u
            plsc.addupdate_scatter(hist_s, [idx_s[pl.ds(t * 16, 16)]], ones16)
            plsc.addupdate_scatter(hist_d, [idx_d[pl.ds(t * 16, 16)]], ones16)
        return carry

    lax.fori_loop(0, EPW // 64, acc, 0)
    pltpu.sync_copy(hist_s, outs_h.at[pl.ds(wid * NP, NP)])
    pltpu.sync_copy(hist_d, outd_h.at[pl.ds(wid * NP, NP)])


@functools.lru_cache(maxsize=None)
def _deg_call():
    return pl.kernel(
        _deg_body,
        out_type=(jax.ShapeDtypeStruct((NW * NP,), jnp.float32),
                  jax.ShapeDtypeStruct((NW * NP,), jnp.float32)),
        mesh=_mesh(),
        scratch_types=[
            pltpu.VMEM((EPW,), jnp.int32),
            pltpu.VMEM((EPW,), jnp.int32),
            pltpu.VMEM((NP,), jnp.float32),
            pltpu.VMEM((NP,), jnp.float32),
        ],
        compiler_params=pltpu.CompilerParams(needs_layout_passes=False),
    )


def _agg_body(h_h, src_i, dst_i, out_h, idx_s, idx_d, rows0, rows1,
              acc, semg0, semg1, sems0, sems1):
    c = lax.axis_index("c")
    s = lax.axis_index("s")
    wid = c * NS + s

    zf = jnp.zeros((16,), jnp.float32)

    def zz(i, carry):
        for cc in range(8):
            rows0[i, pl.ds(cc * 16, 16)] = zf
        return carry

    lax.fori_loop(0, 128, zz, 0)
    for t in range(RPT // 128):
        pltpu.sync_copy(rows0, acc.at[pl.ds(s * RPT + t * 128, 128)])
    plsc.subcore_barrier()

    g0 = rows0.at[pl.ds(0, K)]
    g1 = rows1.at[pl.ds(0, K)]

    def half(hf):
        pltpu.sync_copy(src_i.at[wid, pl.ds(hf * CH2, CH2)], idx_s)
        pltpu.sync_copy(dst_i.at[wid, pl.ds(hf * CH2, CH2)], idx_d)
        pltpu.async_copy(h_h.at[idx_s.at[0]], g0, semg0)

        def pair(i, carry):
            j = 2 * i
            pltpu.async_copy(h_h.at[idx_s.at[j + 1]], g1, semg1)
            pltpu.make_async_copy(h_h.at[idx_s.at[j]], g0, semg0).wait()
            pltpu.sync_copy(g0, acc.at[idx_d.at[j]], add=True)

            @pl.when(j + 2 < CH2)
            def _():
                pltpu.async_copy(h_h.at[idx_s.at[j + 2]], g0, semg0)

            pltpu.make_async_copy(h_h.at[idx_s.at[j + 1]], g1, semg1).wait()
            pltpu.sync_copy(g1, acc.at[idx_d.at[j + 1]], add=True)
            return carry

        lax.fori_loop(0, CH2 // 2, pair, 0)

    half(0)
    half(1)
    plsc.subcore_barrier()
    pltpu.sync_copy(acc.at[pl.ds(s * RPT, RPT)],
                    out_h.at[c, pl.ds(s * RPT, RPT)])


@functools.lru_cache(maxsize=None)
def _agg_call():
    return pl.kernel(
        _agg_body,
        out_type=jax.ShapeDtypeStruct((NC, NP, D), jnp.float32),
        mesh=_mesh(),
        scratch_types=[
            pltpu.VMEM((CH2, K), jnp.int32),
            pltpu.VMEM((CH2, K), jnp.int32),
            pltpu.VMEM((128, D), jnp.float32),
            pltpu.VMEM((128, D), jnp.float32),
            pltpu.VMEM_SHARED((NP, D), jnp.float32),
            pltpu.SemaphoreType.DMA,
            pltpu.SemaphoreType.DMA,
            pltpu.SemaphoreType.DMA,
            pltpu.SemaphoreType.DMA,
        ],
    )


def _norm_body(dsr, ddr, x, ns_o, nd_o, xs_o):
    ones = jnp.ones((NW, 1), jnp.float32)
    dn = (((0,), (0,)), ((), ()))
    deg_s = lax.dot_general(dsr[...], ones, dn, preferred_element_type=jnp.float32)
    deg_d = lax.dot_general(ddr[...], ones, dn, preferred_element_type=jnp.float32)
    n_s = lax.rsqrt(jnp.maximum(deg_s, 1.0))
    n_d = lax.rsqrt(jnp.maximum(deg_d, 1.0))
    ns_o[...] = n_s
    nd_o[...] = n_d
    xs_o[...] = x[...] * n_s


def _norm_call(degs, degd, x):
    deg_spec = pl.BlockSpec((NW, BN), lambda i: (0, i))
    n_spec = pl.BlockSpec((BN, 1), lambda i: (i, 0))
    return pl.pallas_call(
        _norm_body,
        grid=(NB,),
        in_specs=[deg_spec, deg_spec,
                  pl.BlockSpec((BN, D), lambda i: (i, 0))],
        out_specs=[n_spec, n_spec, pl.BlockSpec((BN, D), lambda i: (i, 0))],
        out_shape=[jax.ShapeDtypeStruct((NP, 1), jnp.float32),
                   jax.ShapeDtypeStruct((NP, 1), jnp.float32),
                   jax.ShapeDtypeStruct((NP, D), jnp.float32)],
    )(degs, degd, x)


_P0_SPEC = pl.BlockSpec((1, BN, D), lambda i: (0, i, 0))
_P1_SPEC = pl.BlockSpec((1, BN, D), lambda i: (1, i, 0))


def _layer_body(p0, p1, nd, ns, w, b, o):
    agg = (p0[0] + p1[0]) * nd[...]
    h = jnp.dot(agg, w[...], preferred_element_type=jnp.float32) + b[...]
    o[...] = jnp.maximum(h, 0.0) * ns[...]


def _layer_call(p, nd, ns, w, b):
    blk = pl.BlockSpec((BN, D), lambda i: (i, 0))
    n_spec = pl.BlockSpec((BN, 1), lambda i: (i, 0))
    return pl.pallas_call(
        _layer_body,
        grid=(NB,),
        in_specs=[_P0_SPEC, _P1_SPEC, n_spec, n_spec,
                  pl.BlockSpec((D, D), lambda i: (0, 0)),
                  pl.BlockSpec((1, D), lambda i: (0, 0))],
        out_specs=blk,
        out_shape=jax.ShapeDtypeStruct((NP, D), jnp.float32),
    )(p, p, nd, ns, w, b)


def _head_body(p0, p1, nd, w, b, wp, bp, o, acc):
    i = pl.program_id(0)
    agg = (p0[0] + p1[0]) * nd[...]
    h = jnp.dot(agg, w[...], preferred_element_type=jnp.float32) + b[...]
    h = jnp.maximum(h, 0.0)
    row = lax.broadcasted_iota(jnp.int32, (BN, 1), 0) + i * BN
    h = jnp.where(row < N, h, 0.0)
    colsum = jnp.sum(h, axis=0, keepdims=True)

    @pl.when(i == 0)
    def _():
        acc[...] = colsum

    @pl.when(i > 0)
    def _():
        acc[...] = acc[...] + colsum

    @pl.when(i == NB - 1)
    def _():
        hg = acc[...] * (1.0 / N)
        o[...] = jnp.dot(hg, wp[...], preferred_element_type=jnp.float32) + bp[...]


def _head_call(p, nd, w, b, wpT, bp):
    return pl.pallas_call(
        _head_body,
        grid=(NB,),
        in_specs=[_P0_SPEC, _P1_SPEC, pl.BlockSpec((BN, 1), lambda i: (i, 0)),
                  pl.BlockSpec((D, D), lambda i: (0, 0)),
                  pl.BlockSpec((1, D), lambda i: (0, 0)),
                  pl.BlockSpec((D, C), lambda i: (0, 0)),
                  pl.BlockSpec((1, C), lambda i: (0, 0))],
        out_specs=pl.BlockSpec((1, C), lambda i: (0, 0)),
        out_shape=jax.ShapeDtypeStruct((1, C), jnp.float32),
        scratch_shapes=[pltpu.VMEM((1, D), jnp.float32)],
    )(p, p, nd, w, b, wpT, bp)


def kernel(features, edge_index, W1, b1, W2, b2, W3, b3, Wp, bp):
    src3 = edge_index[0].reshape(NW, CH, K)
    dst3 = edge_index[1].reshape(NW, CH, K)
    xpad = jnp.pad(features, ((0, NP - N), (0, 0)))

    degs, degd = _deg_call()(edge_index[0], edge_index[1])
    norm_src, norm_dst, h = _norm_call(degs.reshape(NW, NP),
                                       degd.reshape(NW, NP), xpad)
    for w, b in ((W1, b1), (W2, b2)):
        p = _agg_call()(h, src3, dst3)
        h = _layer_call(p, norm_dst, norm_src, w, b.reshape(1, D))
    p = _agg_call()(h, src3, dst3)
    return _head_call(p, norm_dst, W3, b3.reshape(1, D),
                      Wp.T, bp.reshape(1, C))
